# Initial kernel scaffold; baseline (speedup 1.0000x reference)
import jax
import jax.numpy as jnp
from jax.experimental import pallas as pl


def kernel(x, edge_index, edge_weight, W1, W2, W2s, fc_W, fc_b, bn_gamma, bn_beta):
    raise NotImplementedError("write your pallas kernel here")


# speedup vs baseline: 1.4909x; 1.4909x over previous
import functools

import jax
import jax.numpy as jnp
from jax.experimental import pallas as pl
from jax.experimental.pallas import tpu as pltpu

_NEG = 0.01
_EPS = 1e-5


def _leaky(v):
    return jnp.where(v >= 0, v, _NEG * v)


def _mm_body(x_ref, w_ref, o_ref):
    o_ref[...] = jax.lax.dot_general(
        x_ref[...], w_ref[...], (((1,), (0,)), ((), ())),
        preferred_element_type=jnp.float32)


def _matmul(x, w, block_rows):
    m, k = x.shape
    _, n = w.shape
    return pl.pallas_call(
        _mm_body,
        grid=(m // block_rows,),
        in_specs=[
            pl.BlockSpec((block_rows, k), lambda i: (i, 0)),
            pl.BlockSpec((k, n), lambda i: (0, 0)),
        ],
        out_specs=pl.BlockSpec((block_rows, n), lambda i: (i, 0)),
        out_shape=jax.ShapeDtypeStruct((m, n), jnp.float32),
    )(x, w)


def _combine_mm_body(p0_ref, p1_ref, w_ref, o_ref):
    h = _leaky(p0_ref[...] + p1_ref[...])
    o_ref[...] = jax.lax.dot_general(
        h, w_ref[...], (((1,), (0,)), ((), ())),
        preferred_element_type=jnp.float32)


def _combine_matmul(p0, p1, w, block_rows):
    m, k = p0.shape
    _, n = w.shape
    return pl.pallas_call(
        _combine_mm_body,
        grid=(m // block_rows,),
        in_specs=[
            pl.BlockSpec((block_rows, k), lambda i: (i, 0)),
            pl.BlockSpec((block_rows, k), lambda i: (i, 0)),
            pl.BlockSpec((k, n), lambda i: (0, 0)),
        ],
        out_specs=pl.BlockSpec((block_rows, n), lambda i: (i, 0)),
        out_shape=jax.ShapeDtypeStruct((m, n), jnp.float32),
    )(p0, p1, w)


def _head_body(q0_ref, q1_ref, fcw_ref, fcb_ref, g_ref, b_ref,
               mu_ref, lv_ref, xr_ref):
    agg = _leaky(q0_ref[...] + q1_ref[...])
    h2 = agg.shape[1] // 2
    mu = agg[:, :h2]
    mu_ref[...] = mu
    lv_ref[...] = agg[:, h2:]
    h = jax.lax.dot_general(
        mu, fcw_ref[...], (((1,), (0,)), ((), ())),
        preferred_element_type=jnp.float32) + fcb_ref[...]
    scale = 1.0 / jnp.sqrt(1.0 + _EPS)
    xr_ref[...] = g_ref[...] * (h * scale) + b_ref[...]


def _head(q0, q1, fc_w, fc_b, bn_g, bn_b, block_rows):
    m, two_h2 = q0.shape
    h2 = two_h2 // 2
    d = fc_w.shape[1]
    fc_b2 = fc_b.reshape(1, d)
    bn_g2 = bn_g.reshape(1, d)
    bn_b2 = bn_b.reshape(1, d)
    return pl.pallas_call(
        _head_body,
        grid=(m // block_rows,),
        in_specs=[
            pl.BlockSpec((block_rows, two_h2), lambda i: (i, 0)),
            pl.BlockSpec((block_rows, two_h2), lambda i: (i, 0)),
            pl.BlockSpec((h2, d), lambda i: (0, 0)),
            pl.BlockSpec((1, d), lambda i: (0, 0)),
            pl.BlockSpec((1, d), lambda i: (0, 0)),
            pl.BlockSpec((1, d), lambda i: (0, 0)),
        ],
        out_specs=[
            pl.BlockSpec((block_rows, h2), lambda i: (i, 0)),
            pl.BlockSpec((block_rows, h2), lambda i: (i, 0)),
            pl.BlockSpec((block_rows, d), lambda i: (i, 0)),
        ],
        out_shape=[
            jax.ShapeDtypeStruct((m, h2), jnp.float32),
            jax.ShapeDtypeStruct((m, h2), jnp.float32),
            jax.ShapeDtypeStruct((m, d), jnp.float32),
        ],
    )(q0, q1, fc_w, fc_b2, bn_g2, bn_b2)


def _gram_body(zi_ref, z_ref, o_ref):
    o_ref[...] = jax.lax.dot_general(
        zi_ref[...], z_ref[...], (((1,), (1,)), ((), ())),
        preferred_element_type=jnp.float32)


def _gram(z, block_rows):
    m, h2 = z.shape
    return pl.pallas_call(
        _gram_body,
        grid=(m // block_rows,),
        in_specs=[
            pl.BlockSpec((block_rows, h2), lambda i: (i, 0)),
            pl.BlockSpec((m, h2), lambda i: (0, 0)),
        ],
        out_specs=pl.BlockSpec((block_rows, m), lambda i: (i, 0)),
        out_shape=jax.ShapeDtypeStruct((m, m), jnp.float32),
    )(z, z)


def _spmm_placeholder(support, src, dst, w, n_nodes):
    msg = support[src] * w[:, None]
    agg = jax.ops.segment_sum(msg, dst, num_segments=n_nodes)
    return agg


def kernel(x, edge_index, edge_weight, W1, W2, W2s, fc_W, fc_b, bn_gamma, bn_beta):
    n = x.shape[0]
    src = edge_index[0]
    dst = edge_index[1]

    support1 = _matmul(x, W1, block_rows=1000)
    agg1 = _spmm_placeholder(support1, src, dst, edge_weight, n)
    zero1 = jnp.zeros_like(agg1)

    w_cat = jnp.concatenate([W2, W2s], axis=1)
    support_cat = _combine_matmul(agg1, zero1, w_cat, block_rows=1000)

    agg2 = _spmm_placeholder(support_cat, src, dst, edge_weight, n)
    zero2 = jnp.zeros_like(agg2)

    mu, logvar, x_rec = _head(agg2, zero2, fc_W, fc_b, bn_gamma, bn_beta,
                              block_rows=1000)
    adj_rec = _gram(mu, block_rows=400)
    return (adj_rec, mu, logvar, mu, x_rec)


# speedup vs baseline: 8.8035x; 5.9049x over previous
import functools

import jax
import jax.numpy as jnp
from jax import lax
from jax.experimental import pallas as pl
from jax.experimental.pallas import tpu as pltpu
from jax.experimental.pallas import tpu_sc as plsc

_NC = 2
_NS = 16
_NW = _NC * _NS
_LANES = 16

_NEG = 0.01
_EPS = 1e-5


def _leaky(v):
    return jnp.where(v >= 0, v, _NEG * v)


def _mm_body(x_ref, w_ref, o_ref):
    o_ref[...] = jax.lax.dot_general(
        x_ref[...], w_ref[...], (((1,), (0,)), ((), ())),
        preferred_element_type=jnp.float32)


def _matmul(x, w, block_rows):
    m, k = x.shape
    _, n = w.shape
    return pl.pallas_call(
        _mm_body,
        grid=(m // block_rows,),
        in_specs=[
            pl.BlockSpec((block_rows, k), lambda i: (i, 0)),
            pl.BlockSpec((k, n), lambda i: (0, 0)),
        ],
        out_specs=pl.BlockSpec((block_rows, n), lambda i: (i, 0)),
        out_shape=jax.ShapeDtypeStruct((m, n), jnp.float32),
    )(x, w)


def _combine_mm_body(p0_ref, p1_ref, w_ref, o_ref):
    h = _leaky(p0_ref[...] + p1_ref[...])
    o_ref[...] = jax.lax.dot_general(
        h, w_ref[...], (((1,), (0,)), ((), ())),
        preferred_element_type=jnp.float32)


def _combine_matmul(p0, p1, w, block_rows):
    m, k = p0.shape
    _, n = w.shape
    return pl.pallas_call(
        _combine_mm_body,
        grid=(m // block_rows,),
        in_specs=[
            pl.BlockSpec((block_rows, k), lambda i: (i, 0)),
            pl.BlockSpec((block_rows, k), lambda i: (i, 0)),
            pl.BlockSpec((k, n), lambda i: (0, 0)),
        ],
        out_specs=pl.BlockSpec((block_rows, n), lambda i: (i, 0)),
        out_shape=jax.ShapeDtypeStruct((m, n), jnp.float32),
    )(p0, p1, w)


def _head_body(q0_ref, q1_ref, fcw_ref, fcb_ref, g_ref, b_ref,
               mu_ref, lv_ref, xr_ref):
    agg = _leaky(q0_ref[...] + q1_ref[...])
    h2 = agg.shape[1] // 2
    mu = agg[:, :h2]
    mu_ref[...] = mu
    lv_ref[...] = agg[:, h2:]
    h = jax.lax.dot_general(
        mu, fcw_ref[...], (((1,), (0,)), ((), ())),
        preferred_element_type=jnp.float32) + fcb_ref[...]
    scale = 1.0 / jnp.sqrt(1.0 + _EPS)
    xr_ref[...] = g_ref[...] * (h * scale) + b_ref[...]


def _head(q0, q1, fc_w, fc_b, bn_g, bn_b, block_rows):
    m, two_h2 = q0.shape
    h2 = two_h2 // 2
    d = fc_w.shape[1]
    fc_b2 = fc_b.reshape(1, d)
    bn_g2 = bn_g.reshape(1, d)
    bn_b2 = bn_b.reshape(1, d)
    return pl.pallas_call(
        _head_body,
        grid=(m // block_rows,),
        in_specs=[
            pl.BlockSpec((block_rows, two_h2), lambda i: (i, 0)),
            pl.BlockSpec((block_rows, two_h2), lambda i: (i, 0)),
            pl.BlockSpec((h2, d), lambda i: (0, 0)),
            pl.BlockSpec((1, d), lambda i: (0, 0)),
            pl.BlockSpec((1, d), lambda i: (0, 0)),
            pl.BlockSpec((1, d), lambda i: (0, 0)),
        ],
        out_specs=[
            pl.BlockSpec((block_rows, h2), lambda i: (i, 0)),
            pl.BlockSpec((block_rows, h2), lambda i: (i, 0)),
            pl.BlockSpec((block_rows, d), lambda i: (i, 0)),
        ],
        out_shape=[
            jax.ShapeDtypeStruct((m, h2), jnp.float32),
            jax.ShapeDtypeStruct((m, h2), jnp.float32),
            jax.ShapeDtypeStruct((m, d), jnp.float32),
        ],
    )(q0, q1, fc_w, fc_b2, bn_g2, bn_b2)


def _gram_body(zi_ref, z_ref, o_ref):
    o_ref[...] = jax.lax.dot_general(
        zi_ref[...], z_ref[...], (((1,), (1,)), ((), ())),
        preferred_element_type=jnp.float32)


def _gram(z, block_rows):
    m, h2 = z.shape
    return pl.pallas_call(
        _gram_body,
        grid=(m // block_rows,),
        in_specs=[
            pl.BlockSpec((block_rows, h2), lambda i: (i, 0)),
            pl.BlockSpec((m, h2), lambda i: (0, 0)),
        ],
        out_specs=pl.BlockSpec((block_rows, m), lambda i: (i, 0)),
        out_shape=jax.ShapeDtypeStruct((m, m), jnp.float32),
    )(z, z)


def _lane_splat(vec, lane):
    dn = lax.GatherDimensionNumbers(
        offset_dims=(), collapsed_slice_dims=(0,), start_index_map=(0,))
    idx = jnp.full((_LANES, 1), lane, jnp.int32)
    return lax.gather(vec, idx, dn, (1,),
                      mode=lax.GatherScatterMode.PROMISE_IN_BOUNDS)


def _make_spmm(n_pad, n_edges, width):
    ept = n_edges // _NW
    ch = 80
    nch = ept // ch
    assert nch * ch == ept
    rpt = n_pad // _NS
    assert rpt % ch == 0
    nz = rpt // ch
    mesh = plsc.VectorSubcoreMesh(core_axis_name="c", subcore_axis_name="s")

    @functools.partial(
        pl.kernel,
        out_type=jax.ShapeDtypeStruct((_NC, n_pad, width), jnp.float32),
        mesh=mesh,
        scratch_types=[
            pltpu.VMEM((nch, ch), jnp.int32),
            pltpu.VMEM((nch, ch), jnp.int32),
            pltpu.VMEM((nch, ch), jnp.float32),
            pltpu.VMEM((ch, width), jnp.float32),
            pltpu.VMEM_SHARED((n_pad, width), jnp.float32),
            pltpu.SemaphoreType.DMA,
        ],
        compiler_params=pltpu.CompilerParams(use_tc_tiling_on_sc=False),
    )
    def spmm(sup_hbm, src_hbm, dst_hbm, w_hbm, out_hbm,
             src_v, dst_v, w_v, g_v, acc, gsem):
        c = lax.axis_index("c")
        s = lax.axis_index("s")
        wid = s * _NC + c
        pltpu.sync_copy(src_hbm.at[wid], src_v)
        pltpu.sync_copy(dst_hbm.at[wid], dst_v)
        pltpu.sync_copy(w_hbm.at[wid], w_v)
        zeros16 = jnp.zeros((_LANES,), jnp.float32)
        for r in range(ch):
            for j in range(width // _LANES):
                g_v[r, pl.ds(_LANES * j, _LANES)] = zeros16
        for i in range(nz):
            pltpu.sync_copy(g_v, acc.at[pl.ds(s * rpt + i * ch, ch)])
        plsc.subcore_barrier()

        def chunk_body(k, carry):
            pltpu.async_copy(sup_hbm.at[src_v.at[k]], g_v, gsem).wait()
            for t in range(ch // _LANES):
                wvec = w_v[k, pl.ds(_LANES * t, _LANES)]
                for p in range(_LANES):
                    e = t * _LANES + p
                    wsplat = _lane_splat(wvec, p)
                    for j in range(width // _LANES):
                        sl = pl.ds(_LANES * j, _LANES)
                        g_v[e, sl] = g_v[e, sl] * wsplat
            pltpu.sync_copy(g_v, acc.at[dst_v.at[k]], add=True)
            return carry

        lax.fori_loop(0, nch, chunk_body, 0)
        plsc.subcore_barrier()
        pltpu.sync_copy(acc.at[pl.ds(s * rpt, rpt)],
                        out_hbm.at[c, pl.ds(s * rpt, rpt)])

    return spmm


def _spmm_sc(support, srcr, dstr, wr, n_nodes, width):
    e = srcr.size
    n_pad = ((n_nodes + 16 * 80 - 1) // (16 * 80)) * (16 * 80)
    out = _make_spmm(n_pad, e, width)(support, srcr, dstr, wr)
    return out[0, :n_nodes], out[1, :n_nodes]


def kernel(x, edge_index, edge_weight, W1, W2, W2s, fc_W, fc_b, bn_gamma, bn_beta):
    n = x.shape[0]
    e = edge_index.shape[1]
    ch = 80
    nch = e // _NW // ch
    srcr = edge_index[0].reshape(_NW, nch, ch)
    dstr = edge_index[1].reshape(_NW, nch, ch)
    wr = edge_weight.reshape(_NW, nch, ch)

    support1 = _matmul(x, W1, block_rows=1000)
    p0, p1 = _spmm_sc(support1, srcr, dstr, wr, n, 64)

    w_cat = jnp.concatenate([W2, W2s], axis=1)
    support_cat = _combine_matmul(p0, p1, w_cat, block_rows=1000)

    q0, q1 = _spmm_sc(support_cat, srcr, dstr, wr, n, 32)

    mu, logvar, x_rec = _head(q0, q1, fc_W, fc_b, bn_gamma, bn_beta,
                              block_rows=1000)
    adj_rec = _gram(mu, block_rows=400)
    return (adj_rec, mu, logvar, mu, x_rec)


# speedup vs baseline: 10.6149x; 1.2058x over previous
import functools

import jax
import jax.numpy as jnp
from jax import lax
from jax.experimental import pallas as pl
from jax.experimental.pallas import tpu as pltpu
from jax.experimental.pallas import tpu_sc as plsc

_NC = 2
_NS = 16
_NW = _NC * _NS
_LANES = 16

_NEG = 0.01
_EPS = 1e-5


def _leaky(v):
    return jnp.where(v >= 0, v, _NEG * v)


def _mm_body(x_ref, w_ref, o_ref):
    o_ref[...] = jax.lax.dot_general(
        x_ref[...], w_ref[...], (((1,), (0,)), ((), ())),
        preferred_element_type=jnp.float32)


def _matmul(x, w, block_rows):
    m, k = x.shape
    _, n = w.shape
    return pl.pallas_call(
        _mm_body,
        grid=(m // block_rows,),
        in_specs=[
            pl.BlockSpec((block_rows, k), lambda i: (i, 0)),
            pl.BlockSpec((k, n), lambda i: (0, 0)),
        ],
        out_specs=pl.BlockSpec((block_rows, n), lambda i: (i, 0)),
        out_shape=jax.ShapeDtypeStruct((m, n), jnp.float32),
    )(x, w)


def _combine_mm_body(p0_ref, p1_ref, w_ref, o_ref):
    h = _leaky(p0_ref[...] + p1_ref[...])
    o_ref[...] = jax.lax.dot_general(
        h, w_ref[...], (((1,), (0,)), ((), ())),
        preferred_element_type=jnp.float32)


def _combine_matmul(p0, p1, w, block_rows):
    m, k = p0.shape
    _, n = w.shape
    return pl.pallas_call(
        _combine_mm_body,
        grid=(m // block_rows,),
        in_specs=[
            pl.BlockSpec((block_rows, k), lambda i: (i, 0)),
            pl.BlockSpec((block_rows, k), lambda i: (i, 0)),
            pl.BlockSpec((k, n), lambda i: (0, 0)),
        ],
        out_specs=pl.BlockSpec((block_rows, n), lambda i: (i, 0)),
        out_shape=jax.ShapeDtypeStruct((m, n), jnp.float32),
    )(p0, p1, w)


def _head_body(q0_ref, q1_ref, fcw_ref, fcb_ref, g_ref, b_ref,
               mu_ref, lv_ref, xr_ref):
    agg = _leaky(q0_ref[...] + q1_ref[...])
    h2 = agg.shape[1] // 2
    mu = agg[:, :h2]
    mu_ref[...] = mu
    lv_ref[...] = agg[:, h2:]
    h = jax.lax.dot_general(
        mu, fcw_ref[...], (((1,), (0,)), ((), ())),
        preferred_element_type=jnp.float32) + fcb_ref[...]
    scale = 1.0 / jnp.sqrt(1.0 + _EPS)
    xr_ref[...] = g_ref[...] * (h * scale) + b_ref[...]


def _head(q0, q1, fc_w, fc_b, bn_g, bn_b, block_rows):
    m, two_h2 = q0.shape
    h2 = two_h2 // 2
    d = fc_w.shape[1]
    fc_b2 = fc_b.reshape(1, d)
    bn_g2 = bn_g.reshape(1, d)
    bn_b2 = bn_b.reshape(1, d)
    return pl.pallas_call(
        _head_body,
        grid=(m // block_rows,),
        in_specs=[
            pl.BlockSpec((block_rows, two_h2), lambda i: (i, 0)),
            pl.BlockSpec((block_rows, two_h2), lambda i: (i, 0)),
            pl.BlockSpec((h2, d), lambda i: (0, 0)),
            pl.BlockSpec((1, d), lambda i: (0, 0)),
            pl.BlockSpec((1, d), lambda i: (0, 0)),
            pl.BlockSpec((1, d), lambda i: (0, 0)),
        ],
        out_specs=[
            pl.BlockSpec((block_rows, h2), lambda i: (i, 0)),
            pl.BlockSpec((block_rows, h2), lambda i: (i, 0)),
            pl.BlockSpec((block_rows, d), lambda i: (i, 0)),
        ],
        out_shape=[
            jax.ShapeDtypeStruct((m, h2), jnp.float32),
            jax.ShapeDtypeStruct((m, h2), jnp.float32),
            jax.ShapeDtypeStruct((m, d), jnp.float32),
        ],
    )(q0, q1, fc_w, fc_b2, bn_g2, bn_b2)


def _gram_body(zi_ref, z_ref, o_ref):
    o_ref[...] = jax.lax.dot_general(
        zi_ref[...], z_ref[...], (((1,), (1,)), ((), ())),
        preferred_element_type=jnp.float32)


def _gram(z, block_rows):
    m, h2 = z.shape
    return pl.pallas_call(
        _gram_body,
        grid=(m // block_rows,),
        in_specs=[
            pl.BlockSpec((block_rows, h2), lambda i: (i, 0)),
            pl.BlockSpec((m, h2), lambda i: (0, 0)),
        ],
        out_specs=pl.BlockSpec((block_rows, m), lambda i: (i, 0)),
        out_shape=jax.ShapeDtypeStruct((m, m), jnp.float32),
    )(z, z)


def _lane_splat(vec, lane):
    dn = lax.GatherDimensionNumbers(
        offset_dims=(), collapsed_slice_dims=(0,), start_index_map=(0,))
    idx = jnp.full((_LANES, 1), lane, jnp.int32)
    return lax.gather(vec, idx, dn, (1,),
                      mode=lax.GatherScatterMode.PROMISE_IN_BOUNDS)


_CH = 128


def _make_spmm(n_pad, nch, width):
    ch = _CH
    rpt = n_pad // _NS
    zch = ch
    assert rpt % zch == 0
    mesh = plsc.VectorSubcoreMesh(core_axis_name="c", subcore_axis_name="s")

    @functools.partial(
        pl.kernel,
        out_type=jax.ShapeDtypeStruct((_NC, n_pad, width), jnp.float32),
        mesh=mesh,
        scratch_types=[
            pltpu.VMEM((nch, ch), jnp.int32),
            pltpu.VMEM((nch, ch), jnp.int32),
            pltpu.VMEM((nch, ch), jnp.float32),
            pltpu.VMEM((2, ch, width), jnp.float32),
            pltpu.VMEM_SHARED((n_pad, width), jnp.float32),
            pltpu.SemaphoreType.DMA,
            pltpu.SemaphoreType.DMA,
        ],
        compiler_params=pltpu.CompilerParams(use_tc_tiling_on_sc=False),
    )
    def spmm(sup_hbm, src_hbm, dst_hbm, w_hbm, out_hbm,
             src_v, dst_v, w_v, g_v, acc, gsem, ssem):
        c = lax.axis_index("c")
        s = lax.axis_index("s")
        wid = s * _NC + c
        pltpu.sync_copy(src_hbm.at[wid], src_v)
        pltpu.sync_copy(dst_hbm.at[wid], dst_v)
        pltpu.sync_copy(w_hbm.at[wid], w_v)
        zeros16 = jnp.zeros((_LANES,), jnp.float32)
        for r in range(zch):
            for j in range(width // _LANES):
                g_v[0, r, pl.ds(_LANES * j, _LANES)] = zeros16
        zsrc = g_v.at[0].at[pl.ds(0, zch)]
        for i in range(rpt // zch):
            pltpu.sync_copy(zsrc, acc.at[pl.ds(s * rpt + i * zch, zch)])
        plsc.subcore_barrier()

        def issue_gather(k, b):
            pltpu.async_copy(sup_hbm.at[src_v.at[k]], g_v.at[b], gsem)

        def wait_gather(k, b):
            pltpu.make_async_copy(sup_hbm.at[src_v.at[k]], g_v.at[b],
                                  gsem).wait()

        def issue_scatter(k, b):
            pltpu.async_copy(g_v.at[b], acc.at[dst_v.at[k]], ssem, add=True)

        def wait_scatter(k, b):
            pltpu.make_async_copy(g_v.at[b], acc.at[dst_v.at[k]],
                                  ssem).wait()

        issue_gather(0, 0)

        def chunk_body(k, carry):
            b = lax.rem(k, 2)
            wait_gather(k, b)
            for t in range(ch // _LANES):
                wvec = w_v[k, pl.ds(_LANES * t, _LANES)]
                for p in range(_LANES):
                    e = t * _LANES + p
                    wsplat = _lane_splat(wvec, p)
                    for j in range(width // _LANES):
                        sl = pl.ds(_LANES * j, _LANES)
                        g_v[b, e, sl] = g_v[b, e, sl] * wsplat
            issue_scatter(k, b)

            @pl.when(k + 1 < nch)
            def _():
                @pl.when(k >= 1)
                def _():
                    wait_scatter(k - 1, 1 - b)
                issue_gather(k + 1, 1 - b)

            return carry

        lax.fori_loop(0, nch, chunk_body, 0)
        wait_scatter(nch - 1, lax.rem(nch - 1, 2))
        plsc.subcore_barrier()
        pltpu.sync_copy(acc.at[pl.ds(s * rpt, rpt)],
                        out_hbm.at[c, pl.ds(s * rpt, rpt)])

    return spmm


def _spmm_sc(support, srcr, dstr, wr, n_nodes, n_pad, width):
    nch = srcr.shape[1]
    out = _make_spmm(n_pad, nch, width)(support, srcr, dstr, wr)
    return out[0, :n_nodes], out[1, :n_nodes]


def _pad_edges(src, dst, w, n_nodes, n_pad):
    e = src.shape[0]
    ept = e // _NW
    nch = (ept + _CH - 1) // _CH
    pad = nch * _CH - ept
    npad_rows = n_pad - n_nodes
    pad_src = (jnp.arange(_NW * pad, dtype=jnp.int32) % n_nodes
               ).reshape(_NW, pad)
    pad_dst = (n_nodes + jnp.arange(_NW * pad, dtype=jnp.int32) % npad_rows
               ).reshape(_NW, pad)
    pad_w = jnp.zeros((_NW, pad), jnp.float32)
    srcr = jnp.concatenate([src.reshape(_NW, ept), pad_src], axis=1)
    dstr = jnp.concatenate([dst.reshape(_NW, ept), pad_dst], axis=1)
    wr = jnp.concatenate([w.reshape(_NW, ept), pad_w], axis=1)
    shape = (_NW, nch, _CH)
    return srcr.reshape(shape), dstr.reshape(shape), wr.reshape(shape)


def kernel(x, edge_index, edge_weight, W1, W2, W2s, fc_W, fc_b, bn_gamma, bn_beta):
    n = x.shape[0]
    n_pad = ((n + 16 * _CH - 1) // (16 * _CH)) * (16 * _CH)
    srcr, dstr, wr = _pad_edges(edge_index[0], edge_index[1], edge_weight,
                                n, n_pad)

    support1 = _matmul(x, W1, block_rows=1000)
    p0, p1 = _spmm_sc(support1, srcr, dstr, wr, n, n_pad, 64)

    w_cat = jnp.concatenate([W2, W2s], axis=1)
    support_cat = _combine_matmul(p0, p1, w_cat, block_rows=1000)

    q0, q1 = _spmm_sc(support_cat, srcr, dstr, wr, n, n_pad, 32)

    mu, logvar, x_rec = _head(q0, q1, fc_W, fc_b, bn_gamma, bn_beta,
                              block_rows=1000)
    adj_rec = _gram(mu, block_rows=400)
    return (adj_rec, mu, logvar, mu, x_rec)


# speedup vs baseline: 13.0272x; 1.2273x over previous
import functools

import jax
import jax.numpy as jnp
from jax import lax
from jax.experimental import pallas as pl
from jax.experimental.pallas import tpu as pltpu
from jax.experimental.pallas import tpu_sc as plsc

_NC = 2
_NS = 16
_NW = _NC * _NS
_LANES = 16

_NEG = 0.01
_EPS = 1e-5


def _leaky(v):
    return jnp.where(v >= 0, v, _NEG * v)


def _mm_body(x_ref, w_ref, o_ref):
    o_ref[...] = jax.lax.dot_general(
        x_ref[...], w_ref[...], (((1,), (0,)), ((), ())),
        preferred_element_type=jnp.float32)


def _matmul(x, w, block_rows):
    m, k = x.shape
    _, n = w.shape
    return pl.pallas_call(
        _mm_body,
        grid=(m // block_rows,),
        in_specs=[
            pl.BlockSpec((block_rows, k), lambda i: (i, 0)),
            pl.BlockSpec((k, n), lambda i: (0, 0)),
        ],
        out_specs=pl.BlockSpec((block_rows, n), lambda i: (i, 0)),
        out_shape=jax.ShapeDtypeStruct((m, n), jnp.float32),
    )(x, w)


def _combine_mm_body(p0_ref, p1_ref, w_ref, o_ref):
    h = _leaky(p0_ref[...] + p1_ref[...])
    o_ref[...] = jax.lax.dot_general(
        h, w_ref[...], (((1,), (0,)), ((), ())),
        preferred_element_type=jnp.float32)


def _combine_matmul(p0, p1, w, block_rows):
    m, k = p0.shape
    _, n = w.shape
    return pl.pallas_call(
        _combine_mm_body,
        grid=(m // block_rows,),
        in_specs=[
            pl.BlockSpec((block_rows, k), lambda i: (i, 0)),
            pl.BlockSpec((block_rows, k), lambda i: (i, 0)),
            pl.BlockSpec((k, n), lambda i: (0, 0)),
        ],
        out_specs=pl.BlockSpec((block_rows, n), lambda i: (i, 0)),
        out_shape=jax.ShapeDtypeStruct((m, n), jnp.float32),
    )(p0, p1, w)


def _head_body(q0_ref, q1_ref, fcw_ref, fcb_ref, g_ref, b_ref,
               mu_ref, lv_ref, xr_ref):
    agg = _leaky(q0_ref[...] + q1_ref[...])
    h2 = agg.shape[1] // 2
    mu = agg[:, :h2]
    mu_ref[...] = mu
    lv_ref[...] = agg[:, h2:]
    h = jax.lax.dot_general(
        mu, fcw_ref[...], (((1,), (0,)), ((), ())),
        preferred_element_type=jnp.float32) + fcb_ref[...]
    scale = 1.0 / jnp.sqrt(1.0 + _EPS)
    xr_ref[...] = g_ref[...] * (h * scale) + b_ref[...]


def _head(q0, q1, fc_w, fc_b, bn_g, bn_b, block_rows):
    m, two_h2 = q0.shape
    h2 = two_h2 // 2
    d = fc_w.shape[1]
    fc_b2 = fc_b.reshape(1, d)
    bn_g2 = bn_g.reshape(1, d)
    bn_b2 = bn_b.reshape(1, d)
    return pl.pallas_call(
        _head_body,
        grid=(m // block_rows,),
        in_specs=[
            pl.BlockSpec((block_rows, two_h2), lambda i: (i, 0)),
            pl.BlockSpec((block_rows, two_h2), lambda i: (i, 0)),
            pl.BlockSpec((h2, d), lambda i: (0, 0)),
            pl.BlockSpec((1, d), lambda i: (0, 0)),
            pl.BlockSpec((1, d), lambda i: (0, 0)),
            pl.BlockSpec((1, d), lambda i: (0, 0)),
        ],
        out_specs=[
            pl.BlockSpec((block_rows, h2), lambda i: (i, 0)),
            pl.BlockSpec((block_rows, h2), lambda i: (i, 0)),
            pl.BlockSpec((block_rows, d), lambda i: (i, 0)),
        ],
        out_shape=[
            jax.ShapeDtypeStruct((m, h2), jnp.float32),
            jax.ShapeDtypeStruct((m, h2), jnp.float32),
            jax.ShapeDtypeStruct((m, d), jnp.float32),
        ],
    )(q0, q1, fc_w, fc_b2, bn_g2, bn_b2)


def _gram_body(zi_ref, z_ref, o_ref):
    o_ref[...] = jax.lax.dot_general(
        zi_ref[...], z_ref[...], (((1,), (1,)), ((), ())),
        preferred_element_type=jnp.float32)


def _gram(z, block_rows):
    m, h2 = z.shape
    return pl.pallas_call(
        _gram_body,
        grid=(m // block_rows,),
        in_specs=[
            pl.BlockSpec((block_rows, h2), lambda i: (i, 0)),
            pl.BlockSpec((m, h2), lambda i: (0, 0)),
        ],
        out_specs=pl.BlockSpec((block_rows, m), lambda i: (i, 0)),
        out_shape=jax.ShapeDtypeStruct((m, m), jnp.float32),
    )(z, z)


def _lane_splat(vec, lane):
    dn = lax.GatherDimensionNumbers(
        offset_dims=(), collapsed_slice_dims=(0,), start_index_map=(0,))
    idx = jnp.full((_LANES, 1), lane, jnp.int32)
    return lax.gather(vec, idx, dn, (1,),
                      mode=lax.GatherScatterMode.PROMISE_IN_BOUNDS)


_CH = 128


def _make_spmm(n_pad, nch, width):
    ch = _CH
    rpt = n_pad // _NS
    zch = ch
    assert rpt % zch == 0
    mesh = plsc.VectorSubcoreMesh(core_axis_name="c", subcore_axis_name="s")

    @functools.partial(
        pl.kernel,
        out_type=jax.ShapeDtypeStruct((_NC, n_pad, width), jnp.float32),
        mesh=mesh,
        scratch_types=[
            pltpu.VMEM((nch, ch), jnp.int32),
            pltpu.VMEM((nch, ch), jnp.int32),
            pltpu.VMEM((nch, ch), jnp.float32),
            pltpu.VMEM((4, ch, width), jnp.float32),
            pltpu.VMEM_SHARED((n_pad, width), jnp.float32),
            pltpu.SemaphoreType.DMA,
            pltpu.SemaphoreType.DMA,
        ],
        compiler_params=pltpu.CompilerParams(use_tc_tiling_on_sc=False),
    )
    def spmm(sup_hbm, src_hbm, dst_hbm, w_hbm, out_hbm,
             src_v, dst_v, w_v, g_v, acc, gsem, ssem):
        c = lax.axis_index("c")
        s = lax.axis_index("s")
        wid = s * _NC + c
        pltpu.sync_copy(src_hbm.at[wid], src_v)
        pltpu.sync_copy(dst_hbm.at[wid], dst_v)
        pltpu.sync_copy(w_hbm.at[wid], w_v)
        zeros16 = jnp.zeros((_LANES,), jnp.float32)
        for r in range(zch):
            for j in range(width // _LANES):
                g_v[0, r, pl.ds(_LANES * j, _LANES)] = zeros16
        zsrc = g_v.at[0].at[pl.ds(0, zch)]
        for i in range(rpt // zch):
            pltpu.sync_copy(zsrc, acc.at[pl.ds(s * rpt + i * zch, zch)])
        plsc.subcore_barrier()

        def issue_gather(k, b):
            pltpu.async_copy(sup_hbm.at[src_v.at[k]], g_v.at[b], gsem)

        def wait_gather(k, b):
            pltpu.make_async_copy(sup_hbm.at[src_v.at[k]], g_v.at[b],
                                  gsem).wait()

        def issue_scatter(k, b):
            pltpu.async_copy(g_v.at[b], acc.at[dst_v.at[k]], ssem, add=True)

        def wait_scatter(k, b):
            pltpu.make_async_copy(g_v.at[b], acc.at[dst_v.at[k]],
                                  ssem).wait()

        assert nch >= 4
        issue_gather(0, 0)

        def chunk_body(k, carry):
            b = lax.rem(k, 4)
            bn = lax.rem(k + 1, 4)

            @pl.when(k >= 3)
            def _():
                wait_scatter(k - 3, bn)

            @pl.when(k + 1 < nch)
            def _():
                issue_gather(k + 1, bn)

            wait_gather(k, b)
            for t in range(ch // _LANES):
                wvec = w_v[k, pl.ds(_LANES * t, _LANES)]
                for p in range(_LANES):
                    e = t * _LANES + p
                    wsplat = _lane_splat(wvec, p)
                    for j in range(width // _LANES):
                        sl = pl.ds(_LANES * j, _LANES)
                        g_v[b, e, sl] = g_v[b, e, sl] * wsplat
            issue_scatter(k, b)
            return carry

        lax.fori_loop(0, nch, chunk_body, 0)
        for d in (3, 2, 1):
            wait_scatter(nch - d, lax.rem(nch - d, 4))
        plsc.subcore_barrier()
        pltpu.sync_copy(acc.at[pl.ds(s * rpt, rpt)],
                        out_hbm.at[c, pl.ds(s * rpt, rpt)])

    return spmm


def _spmm_sc(support, srcr, dstr, wr, n_nodes, n_pad, width):
    nch = srcr.shape[1]
    out = _make_spmm(n_pad, nch, width)(support, srcr, dstr, wr)
    return out[0, :n_nodes], out[1, :n_nodes]


def _pad_edges(src, dst, w, n_nodes, n_pad):
    e = src.shape[0]
    ept = e // _NW
    nch = (ept + _CH - 1) // _CH
    pad = nch * _CH - ept
    npad_rows = n_pad - n_nodes
    pad_src = (jnp.arange(_NW * pad, dtype=jnp.int32) % n_nodes
               ).reshape(_NW, pad)
    pad_dst = (n_nodes + jnp.arange(_NW * pad, dtype=jnp.int32) % npad_rows
               ).reshape(_NW, pad)
    pad_w = jnp.zeros((_NW, pad), jnp.float32)
    srcr = jnp.concatenate([src.reshape(_NW, ept), pad_src], axis=1)
    dstr = jnp.concatenate([dst.reshape(_NW, ept), pad_dst], axis=1)
    wr = jnp.concatenate([w.reshape(_NW, ept), pad_w], axis=1)
    shape = (_NW, nch, _CH)
    return srcr.reshape(shape), dstr.reshape(shape), wr.reshape(shape)


def kernel(x, edge_index, edge_weight, W1, W2, W2s, fc_W, fc_b, bn_gamma, bn_beta):
    n = x.shape[0]
    n_pad = ((n + 16 * _CH - 1) // (16 * _CH)) * (16 * _CH)
    srcr, dstr, wr = _pad_edges(edge_index[0], edge_index[1], edge_weight,
                                n, n_pad)

    support1 = _matmul(x, W1, block_rows=1000)
    p0, p1 = _spmm_sc(support1, srcr, dstr, wr, n, n_pad, 64)

    w_cat = jnp.concatenate([W2, W2s], axis=1)
    support_cat = _combine_matmul(p0, p1, w_cat, block_rows=1000)

    q0, q1 = _spmm_sc(support_cat, srcr, dstr, wr, n, n_pad, 32)

    mu, logvar, x_rec = _head(q0, q1, fc_W, fc_b, bn_gamma, bn_beta,
                              block_rows=1000)
    adj_rec = _gram(mu, block_rows=400)
    return (adj_rec, mu, logvar, mu, x_rec)


# speedup vs baseline: 13.7260x; 1.0536x over previous
import functools

import jax
import jax.numpy as jnp
from jax import lax
from jax.experimental import pallas as pl
from jax.experimental.pallas import tpu as pltpu
from jax.experimental.pallas import tpu_sc as plsc

_NC = 2
_NS = 16
_NW = _NC * _NS
_LANES = 16

_NEG = 0.01
_EPS = 1e-5


def _leaky(v):
    return jnp.where(v >= 0, v, _NEG * v)


def _mm_body(x_ref, w_ref, o_ref):
    o_ref[...] = jax.lax.dot_general(
        x_ref[...], w_ref[...], (((1,), (0,)), ((), ())),
        preferred_element_type=jnp.float32)


def _matmul(x, w, block_rows):
    m, k = x.shape
    _, n = w.shape
    return pl.pallas_call(
        _mm_body,
        grid=(m // block_rows,),
        in_specs=[
            pl.BlockSpec((block_rows, k), lambda i: (i, 0)),
            pl.BlockSpec((k, n), lambda i: (0, 0)),
        ],
        out_specs=pl.BlockSpec((block_rows, n), lambda i: (i, 0)),
        out_shape=jax.ShapeDtypeStruct((m, n), jnp.float32),
    )(x, w)


def _combine_mm_body(p_ref, w_ref, o_ref):
    h = _leaky(p_ref[0] + p_ref[1])
    o_ref[...] = jax.lax.dot_general(
        h, w_ref[...], (((1,), (0,)), ((), ())),
        preferred_element_type=jnp.float32)


def _combine_matmul(p, w, m, block_rows):
    k = p.shape[2]
    n = w.shape[1]
    return pl.pallas_call(
        _combine_mm_body,
        grid=(m // block_rows,),
        in_specs=[
            pl.BlockSpec((2, block_rows, k), lambda i: (0, i, 0)),
            pl.BlockSpec((k, n), lambda i: (0, 0)),
        ],
        out_specs=pl.BlockSpec((block_rows, n), lambda i: (i, 0)),
        out_shape=jax.ShapeDtypeStruct((m, n), jnp.float32),
    )(p, w)


def _head_body(q_ref, fcw_ref, fcb_ref, g_ref, b_ref,
               mu_ref, lv_ref, xr_ref):
    agg = _leaky(q_ref[0] + q_ref[1])
    h2 = agg.shape[1] // 2
    mu = agg[:, :h2]
    mu_ref[...] = mu
    lv_ref[...] = agg[:, h2:]
    h = jax.lax.dot_general(
        mu, fcw_ref[...], (((1,), (0,)), ((), ())),
        preferred_element_type=jnp.float32) + fcb_ref[...]
    scale = 1.0 / jnp.sqrt(1.0 + _EPS)
    xr_ref[...] = g_ref[...] * (h * scale) + b_ref[...]


def _head(q, fc_w, fc_b, bn_g, bn_b, m, block_rows):
    two_h2 = q.shape[2]
    h2 = two_h2 // 2
    d = fc_w.shape[1]
    fc_b2 = fc_b.reshape(1, d)
    bn_g2 = bn_g.reshape(1, d)
    bn_b2 = bn_b.reshape(1, d)
    return pl.pallas_call(
        _head_body,
        grid=(m // block_rows,),
        in_specs=[
            pl.BlockSpec((2, block_rows, two_h2), lambda i: (0, i, 0)),
            pl.BlockSpec((h2, d), lambda i: (0, 0)),
            pl.BlockSpec((1, d), lambda i: (0, 0)),
            pl.BlockSpec((1, d), lambda i: (0, 0)),
            pl.BlockSpec((1, d), lambda i: (0, 0)),
        ],
        out_specs=[
            pl.BlockSpec((block_rows, h2), lambda i: (i, 0)),
            pl.BlockSpec((block_rows, h2), lambda i: (i, 0)),
            pl.BlockSpec((block_rows, d), lambda i: (i, 0)),
        ],
        out_shape=[
            jax.ShapeDtypeStruct((m, h2), jnp.float32),
            jax.ShapeDtypeStruct((m, h2), jnp.float32),
            jax.ShapeDtypeStruct((m, d), jnp.float32),
        ],
    )(q, fc_w, fc_b2, bn_g2, bn_b2)


def _gram_body(zi_ref, z_ref, o_ref):
    o_ref[...] = jax.lax.dot_general(
        zi_ref[...], z_ref[...], (((1,), (1,)), ((), ())),
        preferred_element_type=jnp.float32)


def _gram(z, block_rows):
    m, h2 = z.shape
    return pl.pallas_call(
        _gram_body,
        grid=(m // block_rows,),
        in_specs=[
            pl.BlockSpec((block_rows, h2), lambda i: (i, 0)),
            pl.BlockSpec((m, h2), lambda i: (0, 0)),
        ],
        out_specs=pl.BlockSpec((block_rows, m), lambda i: (i, 0)),
        out_shape=jax.ShapeDtypeStruct((m, m), jnp.float32),
    )(z, z)


def _lane_splat(vec, lane):
    dn = lax.GatherDimensionNumbers(
        offset_dims=(), collapsed_slice_dims=(0,), start_index_map=(0,))
    idx = jnp.full((_LANES, 1), lane, jnp.int32)
    return lax.gather(vec, idx, dn, (1,),
                      mode=lax.GatherScatterMode.PROMISE_IN_BOUNDS)


_CH = 128


def _make_spmm(n_pad, ept, width):
    ch = _CH
    nfull = ept // ch
    tail = ept - nfull * ch
    assert tail % _LANES == 0 and tail < ch
    rpt = n_pad // _NS
    zch = ch
    assert rpt % zch == 0
    mesh = plsc.VectorSubcoreMesh(core_axis_name="c", subcore_axis_name="s")

    @functools.partial(
        pl.kernel,
        out_type=jax.ShapeDtypeStruct((_NC, n_pad, width), jnp.float32),
        mesh=mesh,
        scratch_types=[
            pltpu.VMEM((ept,), jnp.int32),
            pltpu.VMEM((ept,), jnp.int32),
            pltpu.VMEM((ept,), jnp.float32),
            pltpu.VMEM((nfull, ch), jnp.int32),
            pltpu.VMEM((max(tail, _LANES),), jnp.int32),
            pltpu.VMEM((4, ch, width), jnp.float32),
            pltpu.VMEM_SHARED((n_pad, width), jnp.float32),
            pltpu.SemaphoreType.DMA,
            pltpu.SemaphoreType.DMA,
        ],
        compiler_params=pltpu.CompilerParams(use_tc_tiling_on_sc=False),
    )
    def spmm(sup_hbm, src_hbm, dst_hbm, w_hbm, out_hbm,
             src_v, dst_v, w_v, dst2_v, dtail_v, g_v, acc, gsem, ssem):
        c = lax.axis_index("c")
        s = lax.axis_index("s")
        wid = s * _NC + c
        base = wid * ept
        pltpu.sync_copy(src_hbm.at[pl.ds(base, ept)], src_v)
        pltpu.sync_copy(dst_hbm.at[pl.ds(base, ept)], dst_v)
        pltpu.sync_copy(w_hbm.at[pl.ds(base, ept)], w_v)

        def repack(r, carry):
            for t in range(ch // _LANES):
                dst2_v[r, pl.ds(t * _LANES, _LANES)] = (
                    dst_v[pl.ds(r * ch + t * _LANES, _LANES)])
            return carry

        lax.fori_loop(0, nfull, repack, 0)
        for t in range(tail // _LANES):
            dtail_v[pl.ds(t * _LANES, _LANES)] = (
                dst_v[pl.ds(nfull * ch + t * _LANES, _LANES)])

        zeros16 = jnp.zeros((_LANES,), jnp.float32)
        for r in range(zch):
            for j in range(width // _LANES):
                g_v[0, r, pl.ds(_LANES * j, _LANES)] = zeros16
        zsrc = g_v.at[0].at[pl.ds(0, zch)]
        for i in range(rpt // zch):
            pltpu.sync_copy(zsrc, acc.at[pl.ds(s * rpt + i * zch, zch)])
        plsc.subcore_barrier()

        def issue_gather(k, b):
            pltpu.async_copy(sup_hbm.at[src_v.at[pl.ds(k * ch, ch)]],
                             g_v.at[b], gsem)

        def wait_gather(k, b):
            pltpu.make_async_copy(sup_hbm.at[src_v.at[pl.ds(k * ch, ch)]],
                                  g_v.at[b], gsem).wait()

        def issue_scatter(k, b):
            pltpu.async_copy(g_v.at[b], acc.at[dst2_v.at[k]], ssem, add=True)

        def wait_scatter(k, b):
            pltpu.make_async_copy(g_v.at[b], acc.at[dst2_v.at[k]],
                                  ssem).wait()

        assert nfull >= 4
        issue_gather(0, 0)

        def chunk_body(k, carry):
            b = lax.rem(k, 4)
            bn = lax.rem(k + 1, 4)

            @pl.when(k >= 3)
            def _():
                wait_scatter(k - 3, bn)

            @pl.when(k + 1 < nfull)
            def _():
                issue_gather(k + 1, bn)

            wait_gather(k, b)
            for t in range(ch // _LANES):
                wvec = w_v[pl.ds(k * ch + _LANES * t, _LANES)]
                for p in range(_LANES):
                    e = t * _LANES + p
                    wsplat = _lane_splat(wvec, p)
                    for j in range(width // _LANES):
                        sl = pl.ds(_LANES * j, _LANES)
                        g_v[b, e, sl] = g_v[b, e, sl] * wsplat
            issue_scatter(k, b)
            return carry

        lax.fori_loop(0, nfull, chunk_body, 0)
        for d in (3, 2, 1):
            wait_scatter(nfull - d, lax.rem(nfull - d, 4))

        if tail:
            gsl = g_v.at[0].at[pl.ds(0, tail)]
            pltpu.async_copy(
                sup_hbm.at[src_v.at[pl.ds(nfull * ch, tail)]], gsl,
                gsem).wait()
            for t in range(tail // _LANES):
                wvec = w_v[pl.ds(nfull * ch + _LANES * t, _LANES)]
                for p in range(_LANES):
                    e = t * _LANES + p
                    wsplat = _lane_splat(wvec, p)
                    for j in range(width // _LANES):
                        sl = pl.ds(_LANES * j, _LANES)
                        g_v[0, e, sl] = g_v[0, e, sl] * wsplat
            pltpu.sync_copy(gsl, acc.at[dtail_v], add=True)

        plsc.subcore_barrier()
        pltpu.sync_copy(acc.at[pl.ds(s * rpt, rpt)],
                        out_hbm.at[c, pl.ds(s * rpt, rpt)])

    return spmm


def _spmm_sc(support, src, dst, w, n_pad, width):
    ept = src.shape[0] // _NW
    return _make_spmm(n_pad, ept, width)(support, src, dst, w)


def kernel(x, edge_index, edge_weight, W1, W2, W2s, fc_W, fc_b, bn_gamma, bn_beta):
    n = x.shape[0]
    n_pad = ((n + 16 * _CH - 1) // (16 * _CH)) * (16 * _CH)
    src = edge_index[0]
    dst = edge_index[1]

    support1 = _matmul(x, W1, block_rows=1000)
    p = _spmm_sc(support1, src, dst, edge_weight, n_pad, 64)

    w_cat = jnp.concatenate([W2, W2s], axis=1)
    support_cat = _combine_matmul(p, w_cat, n, block_rows=1000)

    q = _spmm_sc(support_cat, src, dst, edge_weight, n_pad, 32)

    mu, logvar, x_rec = _head(q, fc_W, fc_b, bn_gamma, bn_beta, n,
                              block_rows=1000)
    adj_rec = _gram(mu, block_rows=400)
    return (adj_rec, mu, logvar, mu, x_rec)


# speedup vs baseline: 14.8340x; 1.0807x over previous
import functools

import jax
import jax.numpy as jnp
from jax import lax
from jax.experimental import pallas as pl
from jax.experimental.pallas import tpu as pltpu
from jax.experimental.pallas import tpu_sc as plsc

_NC = 2
_NS = 16
_NW = _NC * _NS
_LANES = 16

_NEG = 0.01
_EPS = 1e-5


def _leaky(v):
    return jnp.where(v >= 0, v, _NEG * v)


def _mm_body(x_ref, w_ref, o_ref):
    o_ref[...] = jax.lax.dot_general(
        x_ref[...], w_ref[...], (((1,), (0,)), ((), ())),
        preferred_element_type=jnp.float32)


def _matmul(x, w, block_rows):
    m, k = x.shape
    _, n = w.shape
    return pl.pallas_call(
        _mm_body,
        grid=(m // block_rows,),
        in_specs=[
            pl.BlockSpec((block_rows, k), lambda i: (i, 0)),
            pl.BlockSpec((k, n), lambda i: (0, 0)),
        ],
        out_specs=pl.BlockSpec((block_rows, n), lambda i: (i, 0)),
        out_shape=jax.ShapeDtypeStruct((m, n), jnp.float32),
    )(x, w)


def _combine_mm_body(p_ref, w_ref, o_ref):
    h = _leaky(p_ref[0] + p_ref[1])
    o_ref[...] = jax.lax.dot_general(
        h, w_ref[...], (((1,), (0,)), ((), ())),
        preferred_element_type=jnp.float32)


def _combine_matmul(p, w, m, block_rows):
    k = p.shape[2]
    n = w.shape[1]
    return pl.pallas_call(
        _combine_mm_body,
        grid=(m // block_rows,),
        in_specs=[
            pl.BlockSpec((2, block_rows, k), lambda i: (0, i, 0)),
            pl.BlockSpec((k, n), lambda i: (0, 0)),
        ],
        out_specs=pl.BlockSpec((block_rows, n), lambda i: (i, 0)),
        out_shape=jax.ShapeDtypeStruct((m, n), jnp.float32),
    )(p, w)


def _head_body(q_ref, fcw_ref, fcb_ref, g_ref, b_ref,
               mu_ref, lv_ref, xr_ref):
    agg = _leaky(q_ref[0] + q_ref[1])
    h2 = agg.shape[1] // 2
    mu = agg[:, :h2]
    mu_ref[...] = mu
    lv_ref[...] = agg[:, h2:]
    h = jax.lax.dot_general(
        mu, fcw_ref[...], (((1,), (0,)), ((), ())),
        preferred_element_type=jnp.float32) + fcb_ref[...]
    scale = 1.0 / jnp.sqrt(1.0 + _EPS)
    xr_ref[...] = g_ref[...] * (h * scale) + b_ref[...]


def _head(q, fc_w, fc_b, bn_g, bn_b, m, block_rows):
    two_h2 = q.shape[2]
    h2 = two_h2 // 2
    d = fc_w.shape[1]
    fc_b2 = fc_b.reshape(1, d)
    bn_g2 = bn_g.reshape(1, d)
    bn_b2 = bn_b.reshape(1, d)
    return pl.pallas_call(
        _head_body,
        grid=(m // block_rows,),
        in_specs=[
            pl.BlockSpec((2, block_rows, two_h2), lambda i: (0, i, 0)),
            pl.BlockSpec((h2, d), lambda i: (0, 0)),
            pl.BlockSpec((1, d), lambda i: (0, 0)),
            pl.BlockSpec((1, d), lambda i: (0, 0)),
            pl.BlockSpec((1, d), lambda i: (0, 0)),
        ],
        out_specs=[
            pl.BlockSpec((block_rows, h2), lambda i: (i, 0)),
            pl.BlockSpec((block_rows, h2), lambda i: (i, 0)),
            pl.BlockSpec((block_rows, d), lambda i: (i, 0)),
        ],
        out_shape=[
            jax.ShapeDtypeStruct((m, h2), jnp.float32),
            jax.ShapeDtypeStruct((m, h2), jnp.float32),
            jax.ShapeDtypeStruct((m, d), jnp.float32),
        ],
    )(q, fc_w, fc_b2, bn_g2, bn_b2)


def _transpose_body(z_ref, zt_ref):
    zt_ref[...] = z_ref[...].T


def _transpose(z):
    m, h2 = z.shape
    return pl.pallas_call(
        _transpose_body,
        in_specs=[pl.BlockSpec((m, h2), lambda: (0, 0))],
        out_specs=pl.BlockSpec((h2, m), lambda: (0, 0)),
        out_shape=jax.ShapeDtypeStruct((h2, m), jnp.float32),
    )(z)


def _gram_body(zi_ref, zt_ref, o_ref):
    o_ref[...] = jax.lax.dot_general(
        zi_ref[...], zt_ref[...], (((1,), (0,)), ((), ())),
        preferred_element_type=jnp.float32)


def _gram(z, zt, block_rows):
    m, h2 = z.shape
    return pl.pallas_call(
        _gram_body,
        grid=(m // block_rows,),
        in_specs=[
            pl.BlockSpec((block_rows, h2), lambda i: (i, 0)),
            pl.BlockSpec((h2, m), lambda i: (0, 0)),
        ],
        out_specs=pl.BlockSpec((block_rows, m), lambda i: (i, 0)),
        out_shape=jax.ShapeDtypeStruct((m, m), jnp.float32),
    )(z, zt)


def _lane_splat(vec, lane):
    dn = lax.GatherDimensionNumbers(
        offset_dims=(), collapsed_slice_dims=(0,), start_index_map=(0,))
    idx = jnp.full((_LANES, 1), lane, jnp.int32)
    return lax.gather(vec, idx, dn, (1,),
                      mode=lax.GatherScatterMode.PROMISE_IN_BOUNDS)


_CH = 128
_NBUF = 6


def _make_spmm(n_pad, ept, width):
    ch = _CH
    nfull = ept // ch
    tail = ept - nfull * ch
    assert tail % _LANES == 0 and tail < ch
    rpt = n_pad // _NS
    zch = ch
    assert rpt % zch == 0
    mesh = plsc.VectorSubcoreMesh(core_axis_name="c", subcore_axis_name="s")

    @functools.partial(
        pl.kernel,
        out_type=jax.ShapeDtypeStruct((_NC, n_pad, width), jnp.float32),
        mesh=mesh,
        scratch_types=[
            pltpu.VMEM((ept,), jnp.int32),
            pltpu.VMEM((ept,), jnp.int32),
            pltpu.VMEM((ept,), jnp.float32),
            pltpu.VMEM((nfull, ch), jnp.int32),
            pltpu.VMEM((max(tail, _LANES),), jnp.int32),
            pltpu.VMEM((_NBUF, ch, width), jnp.float32),
            pltpu.VMEM_SHARED((n_pad, width), jnp.float32),
            pltpu.SemaphoreType.DMA,
            pltpu.SemaphoreType.DMA,
        ],
        compiler_params=pltpu.CompilerParams(use_tc_tiling_on_sc=False),
    )
    def spmm(sup_hbm, src_hbm, dst_hbm, w_hbm, out_hbm,
             src_v, dst_v, w_v, dst2_v, dtail_v, g_v, acc, gsem, ssem):
        c = lax.axis_index("c")
        s = lax.axis_index("s")
        wid = s * _NC + c
        base = wid * ept
        pltpu.sync_copy(src_hbm.at[pl.ds(base, ept)], src_v)
        pltpu.sync_copy(dst_hbm.at[pl.ds(base, ept)], dst_v)
        pltpu.sync_copy(w_hbm.at[pl.ds(base, ept)], w_v)

        def repack(r, carry):
            for t in range(ch // _LANES):
                dst2_v[r, pl.ds(t * _LANES, _LANES)] = (
                    dst_v[pl.ds(r * ch + t * _LANES, _LANES)])
            return carry

        lax.fori_loop(0, nfull, repack, 0)
        for t in range(tail // _LANES):
            dtail_v[pl.ds(t * _LANES, _LANES)] = (
                dst_v[pl.ds(nfull * ch + t * _LANES, _LANES)])

        zeros16 = jnp.zeros((_LANES,), jnp.float32)
        for r in range(zch):
            for j in range(width // _LANES):
                g_v[0, r, pl.ds(_LANES * j, _LANES)] = zeros16
        zsrc = g_v.at[0].at[pl.ds(0, zch)]
        for i in range(rpt // zch):
            pltpu.sync_copy(zsrc, acc.at[pl.ds(s * rpt + i * zch, zch)])
        plsc.subcore_barrier()

        def issue_gather(k, b):
            pltpu.async_copy(sup_hbm.at[src_v.at[pl.ds(k * ch, ch)]],
                             g_v.at[b], gsem)

        def wait_gather(k, b):
            pltpu.make_async_copy(sup_hbm.at[src_v.at[pl.ds(k * ch, ch)]],
                                  g_v.at[b], gsem).wait()

        def issue_scatter(k, b):
            pltpu.async_copy(g_v.at[b], acc.at[dst2_v.at[k]], ssem, add=True)

        def wait_scatter(k, b):
            pltpu.make_async_copy(g_v.at[b], acc.at[dst2_v.at[k]],
                                  ssem).wait()

        nb = _NBUF
        pf = nb // 2
        assert nfull >= nb
        for k0 in range(pf):
            issue_gather(k0, k0)

        def chunk_body(k, carry):
            b = lax.rem(k, nb)

            @pl.when(k >= nb - pf)
            def _():
                wait_scatter(k - (nb - pf), 0)

            @pl.when(k + pf < nfull)
            def _():
                issue_gather(k + pf, lax.rem(k + pf, nb))

            wait_gather(k, b)
            for t in range(ch // _LANES):
                wvec = w_v[pl.ds(k * ch + _LANES * t, _LANES)]
                for p in range(_LANES):
                    e = t * _LANES + p
                    wsplat = _lane_splat(wvec, p)
                    for j in range(width // _LANES):
                        sl = pl.ds(_LANES * j, _LANES)
                        g_v[b, e, sl] = g_v[b, e, sl] * wsplat
            issue_scatter(k, b)
            return carry

        lax.fori_loop(0, nfull, chunk_body, 0)
        for d in range(nb - pf, 0, -1):
            wait_scatter(nfull - d, 0)

        if tail:
            gsl = g_v.at[0].at[pl.ds(0, tail)]
            pltpu.async_copy(
                sup_hbm.at[src_v.at[pl.ds(nfull * ch, tail)]], gsl,
                gsem).wait()
            for t in range(tail // _LANES):
                wvec = w_v[pl.ds(nfull * ch + _LANES * t, _LANES)]
                for p in range(_LANES):
                    e = t * _LANES + p
                    wsplat = _lane_splat(wvec, p)
                    for j in range(width // _LANES):
                        sl = pl.ds(_LANES * j, _LANES)
                        g_v[0, e, sl] = g_v[0, e, sl] * wsplat
            pltpu.sync_copy(gsl, acc.at[dtail_v], add=True)

        plsc.subcore_barrier()
        pltpu.sync_copy(acc.at[pl.ds(s * rpt, rpt)],
                        out_hbm.at[c, pl.ds(s * rpt, rpt)])

    return spmm


def _spmm_sc(support, src, dst, w, n_pad, width):
    ept = src.shape[0] // _NW
    return _make_spmm(n_pad, ept, width)(support, src, dst, w)


def kernel(x, edge_index, edge_weight, W1, W2, W2s, fc_W, fc_b, bn_gamma, bn_beta):
    n = x.shape[0]
    n_pad = ((n + 16 * _CH - 1) // (16 * _CH)) * (16 * _CH)
    src = edge_index[0]
    dst = edge_index[1]

    support1 = _matmul(x, W1, block_rows=2000)
    p = _spmm_sc(support1, src, dst, edge_weight, n_pad, 64)

    w_cat = jnp.concatenate([W2, W2s], axis=1)
    support_cat = _combine_matmul(p, w_cat, n, block_rows=2000)

    q = _spmm_sc(support_cat, src, dst, edge_weight, n_pad, 32)

    mu, logvar, x_rec = _head(q, fc_W, fc_b, bn_gamma, bn_beta, n,
                              block_rows=2000)
    zt = _transpose(mu)
    adj_rec = _gram(mu, zt, block_rows=400)
    return (adj_rec, mu, logvar, mu, x_rec)


# speedup vs baseline: 14.9241x; 1.0061x over previous
import functools

import jax
import jax.numpy as jnp
from jax import lax
from jax.experimental import pallas as pl
from jax.experimental.pallas import tpu as pltpu
from jax.experimental.pallas import tpu_sc as plsc

_NC = 2
_NS = 16
_NW = _NC * _NS
_LANES = 16

_NEG = 0.01
_EPS = 1e-5


def _leaky(v):
    return jnp.where(v >= 0, v, _NEG * v)


def _mm_body(x_ref, w_ref, o_ref):
    o_ref[...] = jax.lax.dot_general(
        x_ref[...], w_ref[...], (((1,), (0,)), ((), ())),
        preferred_element_type=jnp.float32)


def _matmul(x, w, block_rows):
    m, k = x.shape
    _, n = w.shape
    return pl.pallas_call(
        _mm_body,
        grid=(m // block_rows,),
        in_specs=[
            pl.BlockSpec((block_rows, k), lambda i: (i, 0)),
            pl.BlockSpec((k, n), lambda i: (0, 0)),
        ],
        out_specs=pl.BlockSpec((block_rows, n), lambda i: (i, 0)),
        out_shape=jax.ShapeDtypeStruct((m, n), jnp.float32),
    )(x, w)


def _combine_mm_body(p_ref, w_ref, o_ref):
    h = _leaky(p_ref[0] + p_ref[1])
    o_ref[...] = jax.lax.dot_general(
        h, w_ref[...], (((1,), (0,)), ((), ())),
        preferred_element_type=jnp.float32)


def _combine_matmul(p, w, m, block_rows):
    k = p.shape[2]
    n = w.shape[1]
    return pl.pallas_call(
        _combine_mm_body,
        grid=(m // block_rows,),
        in_specs=[
            pl.BlockSpec((2, block_rows, k), lambda i: (0, i, 0)),
            pl.BlockSpec((k, n), lambda i: (0, 0)),
        ],
        out_specs=pl.BlockSpec((block_rows, n), lambda i: (i, 0)),
        out_shape=jax.ShapeDtypeStruct((m, n), jnp.float32),
    )(p, w)


def _head_body(q_ref, fcw_ref, fcb_ref, g_ref, b_ref,
               mu_ref, lv_ref, xr_ref):
    agg = _leaky(q_ref[0] + q_ref[1])
    h2 = agg.shape[1] // 2
    mu = agg[:, :h2]
    mu_ref[...] = mu
    lv_ref[...] = agg[:, h2:]
    h = jax.lax.dot_general(
        mu, fcw_ref[...], (((1,), (0,)), ((), ())),
        preferred_element_type=jnp.float32) + fcb_ref[...]
    scale = 1.0 / jnp.sqrt(1.0 + _EPS)
    xr_ref[...] = g_ref[...] * (h * scale) + b_ref[...]


def _head(q, fc_w, fc_b, bn_g, bn_b, m, block_rows):
    two_h2 = q.shape[2]
    h2 = two_h2 // 2
    d = fc_w.shape[1]
    fc_b2 = fc_b.reshape(1, d)
    bn_g2 = bn_g.reshape(1, d)
    bn_b2 = bn_b.reshape(1, d)
    return pl.pallas_call(
        _head_body,
        grid=(m // block_rows,),
        in_specs=[
            pl.BlockSpec((2, block_rows, two_h2), lambda i: (0, i, 0)),
            pl.BlockSpec((h2, d), lambda i: (0, 0)),
            pl.BlockSpec((1, d), lambda i: (0, 0)),
            pl.BlockSpec((1, d), lambda i: (0, 0)),
            pl.BlockSpec((1, d), lambda i: (0, 0)),
        ],
        out_specs=[
            pl.BlockSpec((block_rows, h2), lambda i: (i, 0)),
            pl.BlockSpec((block_rows, h2), lambda i: (i, 0)),
            pl.BlockSpec((block_rows, d), lambda i: (i, 0)),
        ],
        out_shape=[
            jax.ShapeDtypeStruct((m, h2), jnp.float32),
            jax.ShapeDtypeStruct((m, h2), jnp.float32),
            jax.ShapeDtypeStruct((m, d), jnp.float32),
        ],
    )(q, fc_w, fc_b2, bn_g2, bn_b2)


def _gram_body(zi_ref, z_ref, o_ref):
    o_ref[...] = jax.lax.dot_general(
        zi_ref[...], z_ref[...], (((1,), (1,)), ((), ())),
        preferred_element_type=jnp.float32)


def _gram(z, block_rows):
    m, h2 = z.shape
    return pl.pallas_call(
        _gram_body,
        grid=(m // block_rows,),
        in_specs=[
            pl.BlockSpec((block_rows, h2), lambda i: (i, 0)),
            pl.BlockSpec((m, h2), lambda i: (0, 0)),
        ],
        out_specs=pl.BlockSpec((block_rows, m), lambda i: (i, 0)),
        out_shape=jax.ShapeDtypeStruct((m, m), jnp.float32),
    )(z, z)


def _lane_splat(vec, lane):
    dn = lax.GatherDimensionNumbers(
        offset_dims=(), collapsed_slice_dims=(0,), start_index_map=(0,))
    idx = jnp.full((_LANES, 1), lane, jnp.int32)
    return lax.gather(vec, idx, dn, (1,),
                      mode=lax.GatherScatterMode.PROMISE_IN_BOUNDS)


_CH = 128
_NBUF = 6


def _make_spmm(n_pad, ept, width, nbuf):
    ch = _CH
    nfull = ept // ch
    tail = ept - nfull * ch
    assert tail % _LANES == 0 and tail < ch
    rpt = n_pad // _NS
    zch = ch
    assert rpt % zch == 0
    mesh = plsc.VectorSubcoreMesh(core_axis_name="c", subcore_axis_name="s")

    @functools.partial(
        pl.kernel,
        out_type=jax.ShapeDtypeStruct((_NC, n_pad, width), jnp.float32),
        mesh=mesh,
        scratch_types=[
            pltpu.VMEM((ept,), jnp.int32),
            pltpu.VMEM((ept,), jnp.int32),
            pltpu.VMEM((ept,), jnp.float32),
            pltpu.VMEM((nfull, ch), jnp.int32),
            pltpu.VMEM((max(tail, _LANES),), jnp.int32),
            pltpu.VMEM((nbuf, ch, width), jnp.float32),
            pltpu.VMEM_SHARED((n_pad, width), jnp.float32),
            pltpu.SemaphoreType.DMA,
            pltpu.SemaphoreType.DMA,
        ],
        compiler_params=pltpu.CompilerParams(use_tc_tiling_on_sc=False),
    )
    def spmm(sup_hbm, src_hbm, dst_hbm, w_hbm, out_hbm,
             src_v, dst_v, w_v, dst2_v, dtail_v, g_v, acc, gsem, ssem):
        c = lax.axis_index("c")
        s = lax.axis_index("s")
        wid = s * _NC + c
        base = wid * ept
        pltpu.sync_copy(src_hbm.at[pl.ds(base, ept)], src_v)
        pltpu.sync_copy(dst_hbm.at[pl.ds(base, ept)], dst_v)
        pltpu.sync_copy(w_hbm.at[pl.ds(base, ept)], w_v)

        def repack(r, carry):
            for t in range(ch // _LANES):
                dst2_v[r, pl.ds(t * _LANES, _LANES)] = (
                    dst_v[pl.ds(r * ch + t * _LANES, _LANES)])
            return carry

        lax.fori_loop(0, nfull, repack, 0)
        for t in range(tail // _LANES):
            dtail_v[pl.ds(t * _LANES, _LANES)] = (
                dst_v[pl.ds(nfull * ch + t * _LANES, _LANES)])

        zeros16 = jnp.zeros((_LANES,), jnp.float32)
        for r in range(zch):
            for j in range(width // _LANES):
                g_v[0, r, pl.ds(_LANES * j, _LANES)] = zeros16
        zsrc = g_v.at[0].at[pl.ds(0, zch)]
        for i in range(rpt // zch):
            pltpu.sync_copy(zsrc, acc.at[pl.ds(s * rpt + i * zch, zch)])
        plsc.subcore_barrier()

        def issue_gather(k, b):
            pltpu.async_copy(sup_hbm.at[src_v.at[pl.ds(k * ch, ch)]],
                             g_v.at[b], gsem)

        def wait_gather(k, b):
            pltpu.make_async_copy(sup_hbm.at[src_v.at[pl.ds(k * ch, ch)]],
                                  g_v.at[b], gsem).wait()

        def issue_scatter(k, b):
            pltpu.async_copy(g_v.at[b], acc.at[dst2_v.at[k]], ssem, add=True)

        def wait_scatter(k, b):
            pltpu.make_async_copy(g_v.at[b], acc.at[dst2_v.at[k]],
                                  ssem).wait()

        nb = nbuf
        pf = nb // 2
        assert nfull >= nb
        for k0 in range(pf):
            issue_gather(k0, k0)

        def chunk_body(k, carry):
            b = lax.rem(k, nb)

            @pl.when(k >= nb - pf)
            def _():
                wait_scatter(k - (nb - pf), 0)

            @pl.when(k + pf < nfull)
            def _():
                issue_gather(k + pf, lax.rem(k + pf, nb))

            wait_gather(k, b)
            for t in range(ch // _LANES):
                wvec = w_v[pl.ds(k * ch + _LANES * t, _LANES)]
                for p in range(_LANES):
                    e = t * _LANES + p
                    wsplat = _lane_splat(wvec, p)
                    for j in range(width // _LANES):
                        sl = pl.ds(_LANES * j, _LANES)
                        g_v[b, e, sl] = g_v[b, e, sl] * wsplat
            issue_scatter(k, b)
            return carry

        lax.fori_loop(0, nfull, chunk_body, 0)
        for d in range(nb - pf, 0, -1):
            wait_scatter(nfull - d, 0)

        if tail:
            gsl = g_v.at[0].at[pl.ds(0, tail)]
            pltpu.async_copy(
                sup_hbm.at[src_v.at[pl.ds(nfull * ch, tail)]], gsl,
                gsem).wait()
            for t in range(tail // _LANES):
                wvec = w_v[pl.ds(nfull * ch + _LANES * t, _LANES)]
                for p in range(_LANES):
                    e = t * _LANES + p
                    wsplat = _lane_splat(wvec, p)
                    for j in range(width // _LANES):
                        sl = pl.ds(_LANES * j, _LANES)
                        g_v[0, e, sl] = g_v[0, e, sl] * wsplat
            pltpu.sync_copy(gsl, acc.at[dtail_v], add=True)

        plsc.subcore_barrier()
        pltpu.sync_copy(acc.at[pl.ds(s * rpt, rpt)],
                        out_hbm.at[c, pl.ds(s * rpt, rpt)])

    return spmm


def _spmm_sc(support, src, dst, w, n_pad, width, nbuf=_NBUF):
    ept = src.shape[0] // _NW
    return _make_spmm(n_pad, ept, width, nbuf)(support, src, dst, w)


def kernel(x, edge_index, edge_weight, W1, W2, W2s, fc_W, fc_b, bn_gamma, bn_beta):
    n = x.shape[0]
    n_pad = ((n + 16 * _CH - 1) // (16 * _CH)) * (16 * _CH)
    src = edge_index[0]
    dst = edge_index[1]

    support1 = _matmul(x, W1, block_rows=2000)
    p = _spmm_sc(support1, src, dst, edge_weight, n_pad, 64)

    w_cat = jnp.concatenate([W2, W2s], axis=1)
    support_cat = _combine_matmul(p, w_cat, n, block_rows=2000)

    q = _spmm_sc(support_cat, src, dst, edge_weight, n_pad, 32, nbuf=10)

    mu, logvar, x_rec = _head(q, fc_W, fc_b, bn_gamma, bn_beta, n,
                              block_rows=2000)
    adj_rec = _gram(mu, block_rows=400)
    return (adj_rec, mu, logvar, mu, x_rec)


# speedup vs baseline: 15.1790x; 1.0171x over previous
import functools

import jax
import jax.numpy as jnp
from jax import lax
from jax.experimental import pallas as pl
from jax.experimental.pallas import tpu as pltpu
from jax.experimental.pallas import tpu_sc as plsc

_NC = 2
_NS = 16
_NW = _NC * _NS
_LANES = 16

_NEG = 0.01
_EPS = 1e-5


def _leaky(v):
    return jnp.where(v >= 0, v, _NEG * v)


def _mm_body(x_ref, w_ref, o_ref):
    o_ref[...] = jax.lax.dot_general(
        x_ref[...], w_ref[...], (((1,), (0,)), ((), ())),
        preferred_element_type=jnp.float32)


def _matmul(x, w, block_rows):
    m, k = x.shape
    _, n = w.shape
    return pl.pallas_call(
        _mm_body,
        grid=(m // block_rows,),
        in_specs=[
            pl.BlockSpec((block_rows, k), lambda i: (i, 0)),
            pl.BlockSpec((k, n), lambda i: (0, 0)),
        ],
        out_specs=pl.BlockSpec((block_rows, n), lambda i: (i, 0)),
        out_shape=jax.ShapeDtypeStruct((m, n), jnp.float32),
    )(x, w)


def _combine_mm_body(p_ref, w_ref, o_ref):
    h = _leaky(p_ref[0] + p_ref[1])
    o_ref[...] = jax.lax.dot_general(
        h, w_ref[...], (((1,), (0,)), ((), ())),
        preferred_element_type=jnp.float32)


def _combine_matmul(p, w, m, block_rows):
    k = p.shape[2]
    n = w.shape[1]
    return pl.pallas_call(
        _combine_mm_body,
        grid=(m // block_rows,),
        in_specs=[
            pl.BlockSpec((2, block_rows, k), lambda i: (0, i, 0)),
            pl.BlockSpec((k, n), lambda i: (0, 0)),
        ],
        out_specs=pl.BlockSpec((block_rows, n), lambda i: (i, 0)),
        out_shape=jax.ShapeDtypeStruct((m, n), jnp.float32),
    )(p, w)


def _head_body(q_ref, fcw_ref, fcb_ref, g_ref, b_ref,
               mu_ref, lv_ref, xr_ref):
    agg = _leaky(q_ref[0] + q_ref[1])
    h2 = agg.shape[1] // 2
    mu = agg[:, :h2]
    mu_ref[...] = mu
    lv_ref[...] = agg[:, h2:]
    h = jax.lax.dot_general(
        mu, fcw_ref[...], (((1,), (0,)), ((), ())),
        preferred_element_type=jnp.float32) + fcb_ref[...]
    scale = 1.0 / jnp.sqrt(1.0 + _EPS)
    xr_ref[...] = g_ref[...] * (h * scale) + b_ref[...]


def _head(q, fc_w, fc_b, bn_g, bn_b, m, block_rows):
    two_h2 = q.shape[2]
    h2 = two_h2 // 2
    d = fc_w.shape[1]
    fc_b2 = fc_b.reshape(1, d)
    bn_g2 = bn_g.reshape(1, d)
    bn_b2 = bn_b.reshape(1, d)
    return pl.pallas_call(
        _head_body,
        grid=(m // block_rows,),
        in_specs=[
            pl.BlockSpec((2, block_rows, two_h2), lambda i: (0, i, 0)),
            pl.BlockSpec((h2, d), lambda i: (0, 0)),
            pl.BlockSpec((1, d), lambda i: (0, 0)),
            pl.BlockSpec((1, d), lambda i: (0, 0)),
            pl.BlockSpec((1, d), lambda i: (0, 0)),
        ],
        out_specs=[
            pl.BlockSpec((block_rows, h2), lambda i: (i, 0)),
            pl.BlockSpec((block_rows, h2), lambda i: (i, 0)),
            pl.BlockSpec((block_rows, d), lambda i: (i, 0)),
        ],
        out_shape=[
            jax.ShapeDtypeStruct((m, h2), jnp.float32),
            jax.ShapeDtypeStruct((m, h2), jnp.float32),
            jax.ShapeDtypeStruct((m, d), jnp.float32),
        ],
    )(q, fc_w, fc_b2, bn_g2, bn_b2)


def _gram_body(zi_ref, z_ref, o_ref):
    o_ref[...] = jax.lax.dot_general(
        zi_ref[...], z_ref[...], (((1,), (1,)), ((), ())),
        preferred_element_type=jnp.float32)


def _gram(z, block_rows):
    m, h2 = z.shape
    return pl.pallas_call(
        _gram_body,
        grid=(m // block_rows,),
        in_specs=[
            pl.BlockSpec((block_rows, h2), lambda i: (i, 0)),
            pl.BlockSpec((m, h2), lambda i: (0, 0)),
        ],
        out_specs=pl.BlockSpec((block_rows, m), lambda i: (i, 0)),
        out_shape=jax.ShapeDtypeStruct((m, m), jnp.float32),
    )(z, z)


def _lane_splat(vec, lane):
    dn = lax.GatherDimensionNumbers(
        offset_dims=(), collapsed_slice_dims=(0,), start_index_map=(0,))
    idx = jnp.full((_LANES, 1), lane, jnp.int32)
    return lax.gather(vec, idx, dn, (1,),
                      mode=lax.GatherScatterMode.PROMISE_IN_BOUNDS)


_CH = 128
_NBUF = 6


def _make_spmm(n_pad, ept, width, nbuf):
    ch = _CH
    nfull = ept // ch
    tail = ept - nfull * ch
    assert tail % _LANES == 0 and tail < ch
    rpt = n_pad // _NS
    zch = ch
    assert rpt % zch == 0
    mesh = plsc.VectorSubcoreMesh(core_axis_name="c", subcore_axis_name="s")

    @functools.partial(
        pl.kernel,
        out_type=jax.ShapeDtypeStruct((_NC, n_pad, width), jnp.float32),
        mesh=mesh,
        scratch_types=[
            pltpu.VMEM((ept,), jnp.int32),
            pltpu.VMEM((ept,), jnp.int32),
            pltpu.VMEM((ept,), jnp.float32),
            pltpu.VMEM((nfull, ch), jnp.int32),
            pltpu.VMEM((max(tail, _LANES),), jnp.int32),
            pltpu.VMEM((nbuf, ch, width), jnp.float32),
            pltpu.VMEM_SHARED((n_pad, width), jnp.float32),
            pltpu.SemaphoreType.DMA,
            pltpu.SemaphoreType.DMA,
        ],
        compiler_params=pltpu.CompilerParams(use_tc_tiling_on_sc=False),
    )
    def spmm(sup_hbm, src_hbm, dst_hbm, w_hbm, out_hbm,
             src_v, dst_v, w_v, dst2_v, dtail_v, g_v, acc, gsem, ssem):
        c = lax.axis_index("c")
        s = lax.axis_index("s")
        wid = s * _NC + c
        base = wid * ept
        pltpu.sync_copy(src_hbm.at[pl.ds(base, ept)], src_v)
        pltpu.sync_copy(dst_hbm.at[pl.ds(base, ept)], dst_v)
        pltpu.sync_copy(w_hbm.at[pl.ds(base, ept)], w_v)

        def repack(r, carry):
            for t in range(ch // _LANES):
                dst2_v[r, pl.ds(t * _LANES, _LANES)] = (
                    dst_v[pl.ds(r * ch + t * _LANES, _LANES)])
            return carry

        lax.fori_loop(0, nfull, repack, 0)
        for t in range(tail // _LANES):
            dtail_v[pl.ds(t * _LANES, _LANES)] = (
                dst_v[pl.ds(nfull * ch + t * _LANES, _LANES)])

        zeros16 = jnp.zeros((_LANES,), jnp.float32)
        for r in range(zch):
            for j in range(width // _LANES):
                g_v[0, r, pl.ds(_LANES * j, _LANES)] = zeros16
        zsrc = g_v.at[0].at[pl.ds(0, zch)]
        for i in range(rpt // zch):
            pltpu.sync_copy(zsrc, acc.at[pl.ds(s * rpt + i * zch, zch)])
        plsc.subcore_barrier()

        def issue_gather(k, b):
            pltpu.async_copy(sup_hbm.at[src_v.at[pl.ds(k * ch, ch)]],
                             g_v.at[b], gsem)

        def wait_gather(k, b):
            pltpu.make_async_copy(sup_hbm.at[src_v.at[pl.ds(k * ch, ch)]],
                                  g_v.at[b], gsem).wait()

        def issue_scatter(k, b):
            pltpu.async_copy(g_v.at[b], acc.at[dst2_v.at[k]], ssem, add=True)

        def wait_scatter(k, b):
            pltpu.make_async_copy(g_v.at[b], acc.at[dst2_v.at[k]],
                                  ssem).wait()

        nb = nbuf
        pf = nb // 2
        assert nfull >= nb
        for k0 in range(pf):
            issue_gather(k0, k0)

        def chunk_body(k, carry):
            b = lax.rem(k, nb)

            @pl.when(k >= nb - pf)
            def _():
                wait_scatter(k - (nb - pf), 0)

            @pl.when(k + pf < nfull)
            def _():
                issue_gather(k + pf, lax.rem(k + pf, nb))

            wait_gather(k, b)
            for t in range(ch // _LANES):
                wvec = w_v[pl.ds(k * ch + _LANES * t, _LANES)]
                for p in range(_LANES):
                    e = t * _LANES + p
                    wsplat = _lane_splat(wvec, p)
                    for j in range(width // _LANES):
                        sl = pl.ds(_LANES * j, _LANES)
                        g_v[b, e, sl] = g_v[b, e, sl] * wsplat
            issue_scatter(k, b)
            return carry

        lax.fori_loop(0, nfull, chunk_body, 0)
        for d in range(nb - pf, 0, -1):
            wait_scatter(nfull - d, 0)

        if tail:
            gsl = g_v.at[0].at[pl.ds(0, tail)]
            pltpu.async_copy(
                sup_hbm.at[src_v.at[pl.ds(nfull * ch, tail)]], gsl,
                gsem).wait()
            for t in range(tail // _LANES):
                wvec = w_v[pl.ds(nfull * ch + _LANES * t, _LANES)]
                for p in range(_LANES):
                    e = t * _LANES + p
                    wsplat = _lane_splat(wvec, p)
                    for j in range(width // _LANES):
                        sl = pl.ds(_LANES * j, _LANES)
                        g_v[0, e, sl] = g_v[0, e, sl] * wsplat
            pltpu.sync_copy(gsl, acc.at[dtail_v], add=True)

        plsc.subcore_barrier()
        pltpu.sync_copy(acc.at[pl.ds(s * rpt, rpt)],
                        out_hbm.at[c, pl.ds(s * rpt, rpt)])

    return spmm


def _spmm_sc(support, src, dst, w, n_pad, width, nbuf=_NBUF):
    ept = src.shape[0] // _NW
    return _make_spmm(n_pad, ept, width, nbuf)(support, src, dst, w)


def kernel(x, edge_index, edge_weight, W1, W2, W2s, fc_W, fc_b, bn_gamma, bn_beta):
    n = x.shape[0]
    n_pad = ((n + 16 * _CH - 1) // (16 * _CH)) * (16 * _CH)
    src = edge_index[0]
    dst = edge_index[1]

    support1 = _matmul(x, W1, block_rows=5000)
    p = _spmm_sc(support1, src, dst, edge_weight, n_pad, 64)

    w_cat = jnp.concatenate([W2, W2s], axis=1)
    support_cat = _combine_matmul(p, w_cat, n, block_rows=5000)

    q = _spmm_sc(support_cat, src, dst, edge_weight, n_pad, 32, nbuf=10)

    mu, logvar, x_rec = _head(q, fc_W, fc_b, bn_gamma, bn_beta, n,
                              block_rows=5000)
    adj_rec = _gram(mu, block_rows=400)
    return (adj_rec, mu, logvar, mu, x_rec)


# speedup vs baseline: 15.6843x; 1.0333x over previous
import functools

import jax
import jax.numpy as jnp
from jax import lax
from jax.experimental import pallas as pl
from jax.experimental.pallas import tpu as pltpu
from jax.experimental.pallas import tpu_sc as plsc

_NC = 2
_NS = 16
_NW = _NC * _NS
_LANES = 16

_NEG = 0.01
_EPS = 1e-5


def _leaky(v):
    return jnp.where(v >= 0, v, _NEG * v)


def _mm_body(x_ref, w_ref, o_ref):
    o_ref[...] = jax.lax.dot_general(
        x_ref[...], w_ref[...], (((1,), (0,)), ((), ())),
        preferred_element_type=jnp.float32)


def _matmul(x, w, block_rows):
    m, k = x.shape
    _, n = w.shape
    return pl.pallas_call(
        _mm_body,
        grid=(m // block_rows,),
        in_specs=[
            pl.BlockSpec((block_rows, k), lambda i: (i, 0)),
            pl.BlockSpec((k, n), lambda i: (0, 0)),
        ],
        out_specs=pl.BlockSpec((block_rows, n), lambda i: (i, 0)),
        out_shape=jax.ShapeDtypeStruct((m, n), jnp.float32),
    )(x, w)


def _combine_mm_body(p_ref, w_ref, o_ref):
    h = _leaky(p_ref[0] + p_ref[1])
    o_ref[...] = jax.lax.dot_general(
        h, w_ref[...], (((1,), (0,)), ((), ())),
        preferred_element_type=jnp.float32)


def _combine_matmul(p, w, m, block_rows):
    k = p.shape[2]
    n = w.shape[1]
    return pl.pallas_call(
        _combine_mm_body,
        grid=(m // block_rows,),
        in_specs=[
            pl.BlockSpec((2, block_rows, k), lambda i: (0, i, 0)),
            pl.BlockSpec((k, n), lambda i: (0, 0)),
        ],
        out_specs=pl.BlockSpec((block_rows, n), lambda i: (i, 0)),
        out_shape=jax.ShapeDtypeStruct((m, n), jnp.float32),
    )(p, w)


def _head_body(q_ref, fcw_ref, fcb_ref, g_ref, b_ref,
               mu_ref, lv_ref, xr_ref):
    agg = _leaky(q_ref[0] + q_ref[1])
    h2 = agg.shape[1] // 2
    mu = agg[:, :h2]
    mu_ref[...] = mu
    lv_ref[...] = agg[:, h2:]
    h = jax.lax.dot_general(
        mu, fcw_ref[...], (((1,), (0,)), ((), ())),
        preferred_element_type=jnp.float32) + fcb_ref[...]
    scale = 1.0 / jnp.sqrt(1.0 + _EPS)
    xr_ref[...] = g_ref[...] * (h * scale) + b_ref[...]


def _head(q, fc_w, fc_b, bn_g, bn_b, m, block_rows):
    two_h2 = q.shape[2]
    h2 = two_h2 // 2
    d = fc_w.shape[1]
    fc_b2 = fc_b.reshape(1, d)
    bn_g2 = bn_g.reshape(1, d)
    bn_b2 = bn_b.reshape(1, d)
    return pl.pallas_call(
        _head_body,
        grid=(m // block_rows,),
        in_specs=[
            pl.BlockSpec((2, block_rows, two_h2), lambda i: (0, i, 0)),
            pl.BlockSpec((h2, d), lambda i: (0, 0)),
            pl.BlockSpec((1, d), lambda i: (0, 0)),
            pl.BlockSpec((1, d), lambda i: (0, 0)),
            pl.BlockSpec((1, d), lambda i: (0, 0)),
        ],
        out_specs=[
            pl.BlockSpec((block_rows, h2), lambda i: (i, 0)),
            pl.BlockSpec((block_rows, h2), lambda i: (i, 0)),
            pl.BlockSpec((block_rows, d), lambda i: (i, 0)),
        ],
        out_shape=[
            jax.ShapeDtypeStruct((m, h2), jnp.float32),
            jax.ShapeDtypeStruct((m, h2), jnp.float32),
            jax.ShapeDtypeStruct((m, d), jnp.float32),
        ],
    )(q, fc_w, fc_b2, bn_g2, bn_b2)


def _gram_body(zi_ref, z_ref, o_ref):
    o_ref[...] = jax.lax.dot_general(
        zi_ref[...], z_ref[...], (((1,), (1,)), ((), ())),
        preferred_element_type=jnp.float32)


def _gram(z, block_rows):
    m, h2 = z.shape
    return pl.pallas_call(
        _gram_body,
        grid=(m // block_rows,),
        in_specs=[
            pl.BlockSpec((block_rows, h2), lambda i: (i, 0)),
            pl.BlockSpec((m, h2), lambda i: (0, 0)),
        ],
        out_specs=pl.BlockSpec((block_rows, m), lambda i: (i, 0)),
        out_shape=jax.ShapeDtypeStruct((m, m), jnp.float32),
    )(z, z)


def _lane_splat(vec, lane):
    dn = lax.GatherDimensionNumbers(
        offset_dims=(), collapsed_slice_dims=(0,), start_index_map=(0,))
    idx = jnp.full((_LANES, 1), lane, jnp.int32)
    return lax.gather(vec, idx, dn, (1,),
                      mode=lax.GatherScatterMode.PROMISE_IN_BOUNDS)


_CH = 128
_NBUF = 6


def _make_spmm(n_pad, ept, width, nbuf):
    ch = _CH
    nfull = ept // ch
    tail = ept - nfull * ch
    assert tail % _LANES == 0 and tail < ch
    rpt = n_pad // _NS
    zch = ch
    assert rpt % zch == 0
    mesh = plsc.VectorSubcoreMesh(core_axis_name="c", subcore_axis_name="s")

    @functools.partial(
        pl.kernel,
        out_type=jax.ShapeDtypeStruct((_NC, n_pad, width), jnp.float32),
        mesh=mesh,
        scratch_types=[
            pltpu.VMEM((ept,), jnp.int32),
            pltpu.VMEM((ept,), jnp.int32),
            pltpu.VMEM((ept,), jnp.float32),
            pltpu.VMEM((nfull, ch), jnp.int32),
            pltpu.VMEM((max(tail, _LANES),), jnp.int32),
            pltpu.VMEM((nbuf, ch, width), jnp.float32),
            pltpu.VMEM_SHARED((n_pad, width), jnp.float32),
            pltpu.SemaphoreType.DMA,
            pltpu.SemaphoreType.DMA,
        ],
        compiler_params=pltpu.CompilerParams(use_tc_tiling_on_sc=False),
    )
    def spmm(sup_hbm, ei_hbm, w_hbm, out_hbm,
             src_v, dst_v, w_v, dst2_v, dtail_v, g_v, acc, gsem, ssem):
        c = lax.axis_index("c")
        s = lax.axis_index("s")
        wid = s * _NC + c
        base = wid * ept
        pltpu.sync_copy(ei_hbm.at[0, pl.ds(base, ept)], src_v)
        pltpu.sync_copy(ei_hbm.at[1, pl.ds(base, ept)], dst_v)
        pltpu.sync_copy(w_hbm.at[pl.ds(base, ept)], w_v)

        def repack(r, carry):
            for t in range(ch // _LANES):
                dst2_v[r, pl.ds(t * _LANES, _LANES)] = (
                    dst_v[pl.ds(r * ch + t * _LANES, _LANES)])
            return carry

        lax.fori_loop(0, nfull, repack, 0)
        for t in range(tail // _LANES):
            dtail_v[pl.ds(t * _LANES, _LANES)] = (
                dst_v[pl.ds(nfull * ch + t * _LANES, _LANES)])

        zeros16 = jnp.zeros((_LANES,), jnp.float32)
        for r in range(zch):
            for j in range(width // _LANES):
                g_v[0, r, pl.ds(_LANES * j, _LANES)] = zeros16
        zsrc = g_v.at[0].at[pl.ds(0, zch)]
        for i in range(rpt // zch):
            pltpu.sync_copy(zsrc, acc.at[pl.ds(s * rpt + i * zch, zch)])
        plsc.subcore_barrier()

        def issue_gather(k, b):
            pltpu.async_copy(sup_hbm.at[src_v.at[pl.ds(k * ch, ch)]],
                             g_v.at[b], gsem)

        def wait_gather(k, b):
            pltpu.make_async_copy(sup_hbm.at[src_v.at[pl.ds(k * ch, ch)]],
                                  g_v.at[b], gsem).wait()

        def issue_scatter(k, b):
            pltpu.async_copy(g_v.at[b], acc.at[dst2_v.at[k]], ssem, add=True)

        def wait_scatter(k, b):
            pltpu.make_async_copy(g_v.at[b], acc.at[dst2_v.at[k]],
                                  ssem).wait()

        nb = nbuf
        pf = nb // 2
        assert nfull >= nb
        for k0 in range(pf):
            issue_gather(k0, k0)

        def chunk_body(k, carry):
            b = lax.rem(k, nb)

            @pl.when(k >= nb - pf)
            def _():
                wait_scatter(k - (nb - pf), 0)

            @pl.when(k + pf < nfull)
            def _():
                issue_gather(k + pf, lax.rem(k + pf, nb))

            wait_gather(k, b)
            for t in range(ch // _LANES):
                wvec = w_v[pl.ds(k * ch + _LANES * t, _LANES)]
                for p in range(_LANES):
                    e = t * _LANES + p
                    wsplat = _lane_splat(wvec, p)
                    for j in range(width // _LANES):
                        sl = pl.ds(_LANES * j, _LANES)
                        g_v[b, e, sl] = g_v[b, e, sl] * wsplat
            issue_scatter(k, b)
            return carry

        lax.fori_loop(0, nfull, chunk_body, 0)
        for d in range(nb - pf, 0, -1):
            wait_scatter(nfull - d, 0)

        if tail:
            gsl = g_v.at[0].at[pl.ds(0, tail)]
            pltpu.async_copy(
                sup_hbm.at[src_v.at[pl.ds(nfull * ch, tail)]], gsl,
                gsem).wait()
            for t in range(tail // _LANES):
                wvec = w_v[pl.ds(nfull * ch + _LANES * t, _LANES)]
                for p in range(_LANES):
                    e = t * _LANES + p
                    wsplat = _lane_splat(wvec, p)
                    for j in range(width // _LANES):
                        sl = pl.ds(_LANES * j, _LANES)
                        g_v[0, e, sl] = g_v[0, e, sl] * wsplat
            pltpu.sync_copy(gsl, acc.at[dtail_v], add=True)

        plsc.subcore_barrier()
        pltpu.sync_copy(acc.at[pl.ds(s * rpt, rpt)],
                        out_hbm.at[c, pl.ds(s * rpt, rpt)])

    return spmm


def _spmm_sc(support, ei, w, n_pad, width, nbuf=_NBUF):
    ept = ei.shape[1] // _NW
    return _make_spmm(n_pad, ept, width, nbuf)(support, ei, w)


def kernel(x, edge_index, edge_weight, W1, W2, W2s, fc_W, fc_b, bn_gamma, bn_beta):
    n = x.shape[0]
    n_pad = ((n + 16 * _CH - 1) // (16 * _CH)) * (16 * _CH)

    support1 = _matmul(x, W1, block_rows=5000)
    p = _spmm_sc(support1, edge_index, edge_weight, n_pad, 64)

    w_cat = jnp.concatenate([W2, W2s], axis=1)
    support_cat = _combine_matmul(p, w_cat, n, block_rows=5000)

    q = _spmm_sc(support_cat, edge_index, edge_weight, n_pad, 32, nbuf=10)

    mu, logvar, x_rec = _head(q, fc_W, fc_b, bn_gamma, bn_beta, n,
                              block_rows=5000)
    adj_rec = _gram(mu, block_rows=400)
    return (adj_rec, mu, logvar, mu, x_rec)
